# Initial kernel scaffold; baseline (speedup 1.0000x reference)
import jax
import jax.numpy as jnp
from jax.experimental import pallas as pl


def kernel(x, params, batch):
    raise NotImplementedError("write your pallas kernel here")


# speedup vs baseline: 10.9181x; 10.9181x over previous
import functools

import jax
import jax.numpy as jnp
from jax import lax
from jax.experimental import pallas as pl
from jax.experimental.pallas import tpu as pltpu
from jax.experimental.pallas import tpu_sc as plsc

N = 10000
IN = 128
HID = 20
OUT = 10
K = 4
NG = 1024

HP = 32
R = 256
C = 256
NP = 10240
NRB = NP // R
RX = 1024
RE = 512
RP = 512
SENT = 1 << 20
BIG = 1e30

NW = 32
GCH = 128
ROWS_W = K * NP // NW
NCH = ROWS_W // GCH


def _elu(x):
    return jnp.where(x > 0, x, jnp.exp(jnp.minimum(x, 0.0)) - 1.0)


def _dot(a, b):
    return jnp.dot(a.astype(jnp.bfloat16), b.astype(jnp.bfloat16),
                   preferred_element_type=jnp.float32)


def _dot_hi(a, b, dims):
    return lax.dot_general(a, b, dims, precision=jax.lax.Precision.HIGHEST,
                           preferred_element_type=jnp.float32)


def _in_net_kernel(x_ref, w1, b1, w2, b2, w3, b3, o_ref):
    h = _elu(_dot(x_ref[...], w1[...]) + b1[...])
    h = _elu(_dot(h, w2[...]) + b2[...])
    h = _elu(_dot(h, w3[...]) + b3[...])
    o_ref[...] = h


def _full(shape):
    return pl.BlockSpec(shape, lambda i: (0,) * len(shape))


def _in_net(xp, ws):
    return pl.pallas_call(
        _in_net_kernel,
        grid=(NP // RX,),
        in_specs=[pl.BlockSpec((RX, IN), lambda i: (i, 0))]
        + [_full(w.shape) for w in ws],
        out_specs=pl.BlockSpec((RX, HP), lambda i: (i, 0)),
        out_shape=jax.ShapeDtypeStruct((NP, HP), jnp.float32),
    )(xp, *ws)


def _knn_kernel(w0_ref, nt_ref, h_ref, brow_ref, bcol_ref, idx_ref, dist_ref):
    rb = pl.program_id(0)
    r0 = rb * R
    hr = h_ref[pl.ds(r0, R), :]
    hrb = hr.astype(jnp.bfloat16)
    x2r = jnp.sum(hr * hr, axis=1, keepdims=True)
    lane = lax.broadcasted_iota(jnp.int32, (R, HP), 1)
    amat = jnp.where(lane == 0, 1.0, jnp.where(lane == 1, x2r, 0.0))
    br = brow_ref[pl.ds(r0, R), :]
    rid = r0 + lax.broadcasted_iota(jnp.int32, (R, 1), 0)
    w0 = w0_ref[rb]
    nt = nt_ref[rb]

    def body(t, carry):
        bv, bi = carry
        c0 = pl.multiple_of(w0 + t * C, C)
        hc = h_ref[pl.ds(c0, C), :]
        x2c = jnp.sum(hc * hc, axis=1, keepdims=True)
        lanec = lax.broadcasted_iota(jnp.int32, (C, HP), 1)
        bmat = jnp.where(lanec == 0, x2c, jnp.where(lanec == 1, 1.0, 0.0))
        s = _dot_hi(amat, bmat, (((1,), (1,)), ((), ())))
        hh = lax.dot_general(hrb, hc.astype(jnp.bfloat16),
                             (((1,), (1,)), ((), ())),
                             preferred_element_type=jnp.float32)
        d = s - 2.0 * hh
        bc = bcol_ref[:, pl.ds(c0, C)]
        cid = c0 + lax.broadcasted_iota(jnp.int32, (R, C), 1)
        ok = (br == bc) & (cid != rid)
        d = jnp.where(ok, d, BIG)
        tvs, tis = [], []
        for _ in range(K):
            m = jnp.min(d, axis=1, keepdims=True)
            im = jnp.min(jnp.where(d == m, cid, jnp.int32(2**30)),
                         axis=1, keepdims=True)
            tvs.append(m)
            tis.append(im)
            d = jnp.where(cid == im, BIG, d)
        cv = jnp.concatenate([bv] + tvs, axis=1)
        ci = jnp.concatenate([bi] + tis, axis=1)
        pos = lax.broadcasted_iota(jnp.int32, (R, 2 * K), 1)
        nvs, nis = [], []
        for _ in range(K):
            m = jnp.min(cv, axis=1, keepdims=True)
            pm = jnp.min(jnp.where(cv == m, pos, jnp.int32(99)),
                         axis=1, keepdims=True)
            iv = jnp.sum(jnp.where(pos == pm, ci, 0), axis=1, keepdims=True)
            nvs.append(m)
            nis.append(iv)
            cv = jnp.where(pos == pm, BIG, cv)
        return jnp.concatenate(nvs, axis=1), jnp.concatenate(nis, axis=1)

    bv0 = jnp.full((R, K), BIG, jnp.float32)
    bi0 = jnp.zeros((R, K), jnp.int32)
    bv, bi = lax.fori_loop(0, nt, body, (bv0, bi0))
    idx_ref[...] = bi
    dist_ref[...] = bv


def _knn(w0, nt, h, brow, bcol):
    smem = pl.BlockSpec(memory_space=pltpu.MemorySpace.SMEM)
    return pl.pallas_call(
        _knn_kernel,
        grid=(NRB,),
        in_specs=[smem, smem, _full((NP, HP)), _full((NP, 1)), _full((1, NP))],
        out_specs=[pl.BlockSpec((R, K), lambda i: (i, 0)),
                   pl.BlockSpec((R, K), lambda i: (i, 0))],
        out_shape=[jax.ShapeDtypeStruct((NP, K), jnp.int32),
                   jax.ShapeDtypeStruct((NP, K), jnp.float32)],
    )(w0, nt, h, brow, bcol)


def _gather_body(h_hbm, idx_hbm, out_hbm, idx_v, rows_v, sem):
    wid = lax.axis_index("s") * 2 + lax.axis_index("c")
    pltpu.sync_copy(idx_hbm.at[wid], idx_v)
    cps = [pltpu.async_copy(h_hbm.at[idx_v.at[j]], rows_v.at[j], sem)
           for j in range(NCH)]
    for cp in cps:
        cp.wait()
    for j in range(NCH):
        pltpu.sync_copy(rows_v.at[j], out_hbm.at[pl.ds(wid * ROWS_W + j * GCH, GCH)])


@functools.cache
def _sc_gather_fn():
    return pl.kernel(
        _gather_body,
        out_type=jax.ShapeDtypeStruct((K * NP, HP), jnp.float32),
        mesh=plsc.VectorSubcoreMesh(core_axis_name="c", subcore_axis_name="s"),
        compiler_params=pltpu.CompilerParams(use_tc_tiling_on_sc=False),
        scratch_types=[
            pltpu.VMEM((NCH, GCH), jnp.int32),
            pltpu.VMEM((NCH, GCH, HP), jnp.float32),
            pltpu.SemaphoreType.DMA,
        ],
    )


def _sc_gather(h, idx_km):
    return _sc_gather_fn()(h, idx_km)


def _edge_kernel(h_ref, xj_ref, dist_ref, w1, b1, w2, b2, o_ref):
    rb = pl.program_id(0)
    r0 = rb * RE
    xi = h_ref[pl.ds(r0, RE), :]
    acc = jnp.zeros((RE, HP), jnp.float32)
    for k in range(K):
        xjk = xj_ref[pl.ds(k * NP + r0, RE), :]
        cat = jnp.concatenate([xi, xjk - xi], axis=1)
        m1 = _elu(_dot(cat, w1[...]) + b1[...])
        m2 = _elu(_dot(m1, w2[...]) + b2[...])
        vk = (dist_ref[:, k:k + 1] < 1e20).astype(jnp.float32)
        acc = acc + m2 * vk
    o_ref[...] = acc


def _edge(h, xj, dist, ws):
    return pl.pallas_call(
        _edge_kernel,
        grid=(NP // RE,),
        in_specs=[_full((NP, HP)), _full((K * NP, HP)),
                  pl.BlockSpec((RE, K), lambda i: (i, 0))]
        + [_full(w.shape) for w in ws],
        out_specs=pl.BlockSpec((RE, HP), lambda i: (i, 0)),
        out_shape=jax.ShapeDtypeStruct((NP, HP), jnp.float32),
    )(h, xj, dist, *ws)


def _pool_kernel(h_ref, bcol_ref, v1, c1, v2, c2, v3, c3, o_ref, acc_ref):
    rb = pl.program_id(0)
    gids = lax.broadcasted_iota(jnp.int32, (NG, RP), 0)
    s = (gids == bcol_ref[...]).astype(jnp.float32)
    part = _dot_hi(s, h_ref[...], (((1,), (0,)), ((), ())))

    @pl.when(rb == 0)
    def _():
        acc_ref[...] = part

    @pl.when(rb > 0)
    def _():
        acc_ref[...] = acc_ref[...] + part

    @pl.when(rb == NP // RP - 1)
    def _():
        g = acc_ref[...]
        o1 = _elu(_dot(g, v1[...]) + c1[...])
        o2 = _elu(_dot(o1, v2[...]) + c2[...])
        lg = _dot(o2, v3[...]) + c3[...]
        mx = jnp.max(lg, axis=1, keepdims=True)
        ls = jnp.log(jnp.sum(jnp.exp(lg - mx), axis=1, keepdims=True))
        o_ref[...] = lg - mx - ls


def _pool_out(h, bcol, ws):
    return pl.pallas_call(
        _pool_kernel,
        grid=(NP // RP,),
        in_specs=[pl.BlockSpec((RP, HP), lambda i: (i, 0)),
                  pl.BlockSpec((1, RP), lambda i: (0, i))]
        + [_full(w.shape) for w in ws],
        out_specs=pl.BlockSpec((NG, OUT), lambda i: (0, 0)),
        out_shape=jax.ShapeDtypeStruct((NG, OUT), jnp.float32),
        scratch_shapes=[pltpu.VMEM((NG, HP), jnp.float32)],
    )(h, bcol, *ws)


def _pad_w(w, rows, cols):
    return jnp.pad(w, ((0, rows - w.shape[0]), (0, cols - w.shape[1])))


def _pad_b(b, cols):
    return jnp.pad(b, (0, cols - b.shape[0])).reshape(1, cols)


def kernel(x, params, batch):
    batch = batch.astype(jnp.int32)
    xp = jnp.pad(x, ((0, NP - N), (0, 0)))
    bpad = jnp.concatenate([batch, jnp.full((NP - N,), SENT, jnp.int32)])
    brow = bpad.reshape(NP, 1)
    bcol = bpad.reshape(1, NP)

    rbs = jnp.arange(NRB)
    first_rows = jnp.minimum(rbs * R, N - 1)
    last_rows = jnp.minimum(rbs * R + R - 1, N - 1)
    starts = jnp.searchsorted(batch, batch[first_rows], side="left")
    ends = jnp.searchsorted(batch, batch[last_rows], side="right")
    w0 = (starts // C) * C
    nt = (ends - w0 + C - 1) // C
    real = rbs * R < N
    w0 = jnp.where(real, w0, 0).astype(jnp.int32)
    nt = jnp.where(real, nt, 0).astype(jnp.int32)

    p_in = params["in_net"]
    in_ws = []
    fan = [IN, HP, HP]
    for (w, b), f in zip(p_in, fan):
        in_ws += [_pad_w(w, f, HP), _pad_b(b, HP)]

    edge_ws = []
    for (w1, b1), (w2, b2) in params["edge_nets"]:
        w1p = jnp.zeros((2 * HP, HP), jnp.float32)
        w1p = w1p.at[:HID, :HID].set(w1[:HID])
        w1p = w1p.at[HP:HP + HID, :HID].set(w1[HID:])
        edge_ws.append([w1p, _pad_b(b1, HP),
                        _pad_w(w2, HP, HP), _pad_b(b2, HP)])

    p_out = params["out_net"]
    out_ws = []
    for (w, b), c in zip(p_out, [HP, HP, OUT]):
        out_ws += [_pad_w(w, HP, c), _pad_b(b, c)]

    h = _in_net(xp, in_ws)
    for lws in edge_ws:
        idx, dist = _knn(w0, nt, h, brow, bcol)
        idx_km = idx.T.reshape(NW, NCH, GCH)
        xj = _sc_gather(h, idx_km)
        h = _edge(h, xj, dist, lws)
    return _pool_out(h, bcol, out_ws)


# speedup vs baseline: 18.6715x; 1.7101x over previous
import functools

import jax
import jax.numpy as jnp
from jax import lax
from jax.experimental import pallas as pl
from jax.experimental.pallas import tpu as pltpu
from jax.experimental.pallas import tpu_sc as plsc

N = 10000
IN = 128
HID = 20
OUT = 10
K = 4
NG = 1024

HP = 32
R = 128
C = 128
NP = 10240
NRB = NP // R
RX = 1024
RE = 512
RP = 512
SENT = 1 << 20
BIG = 1e30

NW = 32
GCH = 128
ROWS_W = K * NP // NW
NCH = ROWS_W // GCH


def _elu(x):
    return jnp.where(x > 0, x, jnp.exp(jnp.minimum(x, 0.0)) - 1.0)


def _dot(a, b):
    return jnp.dot(a.astype(jnp.bfloat16), b.astype(jnp.bfloat16),
                   preferred_element_type=jnp.float32)


def _dot_hi(a, b, dims):
    return lax.dot_general(a, b, dims, precision=jax.lax.Precision.HIGHEST,
                           preferred_element_type=jnp.float32)


def _in_net_kernel(x_ref, w1, b1, w2, b2, w3, b3, o_ref):
    h = _elu(_dot(x_ref[...], w1[...]) + b1[...])
    h = _elu(_dot(h, w2[...]) + b2[...])
    h = _elu(_dot(h, w3[...]) + b3[...])
    o_ref[...] = h


def _full(shape):
    return pl.BlockSpec(shape, lambda i: (0,) * len(shape))


def _in_net(xp, ws):
    return pl.pallas_call(
        _in_net_kernel,
        grid=(NP // RX,),
        in_specs=[pl.BlockSpec((RX, IN), lambda i: (i, 0))]
        + [_full(w.shape) for w in ws],
        out_specs=pl.BlockSpec((RX, HP), lambda i: (i, 0)),
        out_shape=jax.ShapeDtypeStruct((NP, HP), jnp.float32),
    )(xp, *ws)


def _knn_kernel(w0_ref, nt_ref, h_ref, brow_ref, bcol_ref, idx_ref, val_ref):
    rb = pl.program_id(0)
    r0 = pl.multiple_of(rb * R, R)
    hr = h_ref[pl.ds(r0, R), :]
    hrb = hr.astype(jnp.bfloat16)
    x2rt = _dot_hi(jnp.ones((1, HP), jnp.float32), hr * hr,
                   (((1,), (1,)), ((), ())))
    br = bcol_ref[:, pl.ds(r0, R)]
    rid = r0 + lax.broadcasted_iota(jnp.int32, (1, R), 1)
    w0 = w0_ref[rb]
    nt = nt_ref[rb]

    def body(t, carry):
        bv, bi = carry
        c0 = pl.multiple_of(w0 + t * C, 8)
        hc = h_ref[pl.ds(c0, C), :]
        x2c = jnp.sum(hc * hc, axis=1, keepdims=True)
        hh = lax.dot_general(hc.astype(jnp.bfloat16), hrb,
                             (((1,), (1,)), ((), ())),
                             preferred_element_type=jnp.float32)
        d = (x2c + x2rt) - 2.0 * hh
        bc = brow_ref[pl.ds(c0, C), :]
        cid = c0 + lax.broadcasted_iota(jnp.int32, (C, R), 0)
        ok = (bc == br) & (cid != rid)
        d = jnp.where(ok, d, BIG)
        tvs, tis = [], []
        for _ in range(K):
            m = jnp.min(d, axis=0, keepdims=True)
            im = jnp.min(jnp.where(d == m, cid, jnp.int32(2**30)),
                         axis=0, keepdims=True)
            tvs.append(m)
            tis.append(im)
            d = jnp.where(cid == im, BIG, d)
        cv = jnp.concatenate([bv] + tvs, axis=0)
        ci = jnp.concatenate([bi] + tis, axis=0)
        pos = lax.broadcasted_iota(jnp.int32, (2 * K, R), 0)
        nvs, nis = [], []
        for _ in range(K):
            m = jnp.min(cv, axis=0, keepdims=True)
            pm = jnp.min(jnp.where(cv == m, pos, jnp.int32(99)),
                         axis=0, keepdims=True)
            iv = jnp.sum(jnp.where(pos == pm, ci, 0), axis=0, keepdims=True)
            nvs.append(m)
            nis.append(iv)
            cv = jnp.where(pos == pm, BIG, cv)
        return jnp.concatenate(nvs, axis=0), jnp.concatenate(nis, axis=0)

    bv0 = jnp.full((K, R), BIG, jnp.float32)
    bi0 = jnp.zeros((K, R), jnp.int32)
    bv, bi = lax.fori_loop(0, nt, body, (bv0, bi0))
    idx_ref[...] = bi
    validf = (bv < 1e20).astype(jnp.float32)
    eye = (lax.broadcasted_iota(jnp.int32, (K, K), 0)
           == lax.broadcasted_iota(jnp.int32, (K, K), 1)).astype(jnp.float32)
    val_ref[...] = _dot_hi(validf, eye, (((0,), (0,)), ((), ())))


def _knn(w0, nt, h, brow, bcol):
    smem = pl.BlockSpec(memory_space=pltpu.MemorySpace.SMEM)
    return pl.pallas_call(
        _knn_kernel,
        grid=(NRB,),
        in_specs=[smem, smem, _full((NP, HP)), _full((NP, 1)), _full((1, NP))],
        out_specs=[pl.BlockSpec((K, R), lambda i: (0, i)),
                   pl.BlockSpec((R, K), lambda i: (i, 0))],
        out_shape=[jax.ShapeDtypeStruct((K, NP), jnp.int32),
                   jax.ShapeDtypeStruct((NP, K), jnp.float32)],
    )(w0, nt, h, brow, bcol)


def _gather_body(h_hbm, idx_hbm, out_hbm, idx_v, rows_v, sem):
    wid = lax.axis_index("s") * 2 + lax.axis_index("c")
    pltpu.sync_copy(idx_hbm.at[wid], idx_v)
    cps = [pltpu.async_copy(h_hbm.at[idx_v.at[j]], rows_v.at[j], sem)
           for j in range(NCH)]
    for cp in cps:
        cp.wait()
    for j in range(NCH):
        pltpu.sync_copy(rows_v.at[j], out_hbm.at[pl.ds(wid * ROWS_W + j * GCH, GCH)])


@functools.cache
def _sc_gather_fn():
    return pl.kernel(
        _gather_body,
        out_type=jax.ShapeDtypeStruct((K * NP, HP), jnp.float32),
        mesh=plsc.VectorSubcoreMesh(core_axis_name="c", subcore_axis_name="s"),
        compiler_params=pltpu.CompilerParams(use_tc_tiling_on_sc=False),
        scratch_types=[
            pltpu.VMEM((NCH, GCH), jnp.int32),
            pltpu.VMEM((NCH, GCH, HP), jnp.float32),
            pltpu.SemaphoreType.DMA,
        ],
    )


def _sc_gather(h, idx_km):
    return _sc_gather_fn()(h, idx_km)


def _edge_kernel(h_ref, xj_ref, val_ref, w1, b1, w2, b2, o_ref):
    rb = pl.program_id(0)
    r0 = rb * RE
    xi = h_ref[pl.ds(r0, RE), :]
    acc = jnp.zeros((RE, HP), jnp.float32)
    for k in range(K):
        xjk = xj_ref[pl.ds(k * NP + r0, RE), :]
        cat = jnp.concatenate([xi, xjk - xi], axis=1)
        m1 = _elu(_dot(cat, w1[...]) + b1[...])
        m2 = _elu(_dot(m1, w2[...]) + b2[...])
        vk = val_ref[:, k:k + 1]
        acc = acc + m2 * vk
    o_ref[...] = acc


def _edge(h, xj, val, ws):
    return pl.pallas_call(
        _edge_kernel,
        grid=(NP // RE,),
        in_specs=[_full((NP, HP)), _full((K * NP, HP)),
                  pl.BlockSpec((RE, K), lambda i: (i, 0))]
        + [_full(w.shape) for w in ws],
        out_specs=pl.BlockSpec((RE, HP), lambda i: (i, 0)),
        out_shape=jax.ShapeDtypeStruct((NP, HP), jnp.float32),
    )(h, xj, val, *ws)


def _pool_kernel(h_ref, bcol_ref, v1, c1, v2, c2, v3, c3, o_ref, acc_ref):
    rb = pl.program_id(0)
    gids = lax.broadcasted_iota(jnp.int32, (NG, RP), 0)
    s = (gids == bcol_ref[...]).astype(jnp.float32)
    part = _dot_hi(s, h_ref[...], (((1,), (0,)), ((), ())))

    @pl.when(rb == 0)
    def _():
        acc_ref[...] = part

    @pl.when(rb > 0)
    def _():
        acc_ref[...] = acc_ref[...] + part

    @pl.when(rb == NP // RP - 1)
    def _():
        g = acc_ref[...]
        o1 = _elu(_dot(g, v1[...]) + c1[...])
        o2 = _elu(_dot(o1, v2[...]) + c2[...])
        lg = _dot(o2, v3[...]) + c3[...]
        mx = jnp.max(lg, axis=1, keepdims=True)
        ls = jnp.log(jnp.sum(jnp.exp(lg - mx), axis=1, keepdims=True))
        o_ref[...] = lg - mx - ls


def _pool_out(h, bcol, ws):
    return pl.pallas_call(
        _pool_kernel,
        grid=(NP // RP,),
        in_specs=[pl.BlockSpec((RP, HP), lambda i: (i, 0)),
                  pl.BlockSpec((1, RP), lambda i: (0, i))]
        + [_full(w.shape) for w in ws],
        out_specs=pl.BlockSpec((NG, OUT), lambda i: (0, 0)),
        out_shape=jax.ShapeDtypeStruct((NG, OUT), jnp.float32),
        scratch_shapes=[pltpu.VMEM((NG, HP), jnp.float32)],
    )(h, bcol, *ws)


def _pad_w(w, rows, cols):
    return jnp.pad(w, ((0, rows - w.shape[0]), (0, cols - w.shape[1])))


def _pad_b(b, cols):
    return jnp.pad(b, (0, cols - b.shape[0])).reshape(1, cols)


def kernel(x, params, batch):
    batch = batch.astype(jnp.int32)
    xp = jnp.pad(x, ((0, NP - N), (0, 0)))
    bpad = jnp.concatenate([batch, jnp.full((NP - N,), SENT, jnp.int32)])
    brow = bpad.reshape(NP, 1)
    bcol = bpad.reshape(1, NP)

    rbs = jnp.arange(NRB)
    first_rows = jnp.minimum(rbs * R, N - 1)
    last_rows = jnp.minimum(rbs * R + R - 1, N - 1)
    starts = jnp.searchsorted(batch, batch[first_rows], side="left")
    ends = jnp.searchsorted(batch, batch[last_rows], side="right")
    w0 = (starts // 8) * 8
    nt = (ends - w0 + C - 1) // C
    real = rbs * R < N
    w0 = jnp.where(real, w0, 0).astype(jnp.int32)
    nt = jnp.where(real, nt, 0).astype(jnp.int32)

    p_in = params["in_net"]
    in_ws = []
    fan = [IN, HP, HP]
    for (w, b), f in zip(p_in, fan):
        in_ws += [_pad_w(w, f, HP), _pad_b(b, HP)]

    edge_ws = []
    for (w1, b1), (w2, b2) in params["edge_nets"]:
        w1p = jnp.zeros((2 * HP, HP), jnp.float32)
        w1p = w1p.at[:HID, :HID].set(w1[:HID])
        w1p = w1p.at[HP:HP + HID, :HID].set(w1[HID:])
        edge_ws.append([w1p, _pad_b(b1, HP),
                        _pad_w(w2, HP, HP), _pad_b(b2, HP)])

    p_out = params["out_net"]
    out_ws = []
    for (w, b), c in zip(p_out, [HP, HP, OUT]):
        out_ws += [_pad_w(w, HP, c), _pad_b(b, c)]

    h = _in_net(xp, in_ws)
    for lws in edge_ws:
        idx, val = _knn(w0, nt, h, brow, bcol)
        idx_km = idx.reshape(NW, NCH, GCH)
        xj = _sc_gather(h, idx_km)
        h = _edge(h, xj, val, lws)
    return _pool_out(h, bcol, out_ws)


# speedup vs baseline: 19.4351x; 1.0409x over previous
import functools

import jax
import jax.numpy as jnp
from jax import lax
from jax.experimental import pallas as pl
from jax.experimental.pallas import tpu as pltpu
from jax.experimental.pallas import tpu_sc as plsc

N = 10000
IN = 128
HID = 20
OUT = 10
K = 4
NG = 1024

HP = 32
R = 128
C = 128
NP = 10240
NRB = NP // R
RX = 1024
RE = 512
RP = 512
SENT = 1 << 20
BIG = 1e30

NW = 32
GCH = 128


def _elu(x):
    return jnp.where(x > 0, x, jnp.exp(jnp.minimum(x, 0.0)) - 1.0)


def _dot(a, b):
    return jnp.dot(a.astype(jnp.bfloat16), b.astype(jnp.bfloat16),
                   preferred_element_type=jnp.float32)


def _dot_hi(a, b, dims):
    return lax.dot_general(a, b, dims, precision=jax.lax.Precision.HIGHEST,
                           preferred_element_type=jnp.float32)


def _in_net_kernel(x_ref, w1, b1, w2, b2, w3, b3, o_ref):
    h = _elu(_dot(x_ref[...], w1[...]) + b1[...])
    h = _elu(_dot(h, w2[...]) + b2[...])
    h = _elu(_dot(h, w3[...]) + b3[...])
    o_ref[...] = h


def _full(shape):
    return pl.BlockSpec(shape, lambda i: (0,) * len(shape))


def _in_net(xp, ws):
    return pl.pallas_call(
        _in_net_kernel,
        grid=(NP // RX,),
        in_specs=[pl.BlockSpec((RX, IN), lambda i: (i, 0))]
        + [_full(w.shape) for w in ws],
        out_specs=pl.BlockSpec((RX, HP), lambda i: (i, 0)),
        out_shape=jax.ShapeDtypeStruct((NP, HP), jnp.float32),
    )(xp, *ws)


def _win_kernel(brs_ref, bcol_ref, w0_ref, nt_ref):
    bf = brs_ref[:, 0:1]
    bl = brs_ref[:, R - 1:R]
    bc = bcol_ref[...]
    starts = jnp.sum((bc < bf).astype(jnp.int32), axis=1, keepdims=True)
    ends = jnp.sum((bc <= bl).astype(jnp.int32), axis=1, keepdims=True)
    w0 = (starts // 8) * 8
    w0_ref[...] = w0
    nt_ref[...] = jnp.minimum((ends - w0 + C - 1) // C, (NP - w0) // C)


def _windows(brs, bcol):
    return pl.pallas_call(
        _win_kernel,
        grid=(1,),
        in_specs=[_full((NRB, R)), _full((1, NP))],
        out_specs=[pl.BlockSpec((NRB, 1), lambda i: (0, 0)),
                   pl.BlockSpec((NRB, 1), lambda i: (0, 0))],
        out_shape=[jax.ShapeDtypeStruct((NRB, 1), jnp.int32),
                   jax.ShapeDtypeStruct((NRB, 1), jnp.int32)],
    )(brs, bcol)


def _knn_kernel(w0_ref, nt_ref, h_ref, brow_ref, bcol_ref, idx_ref, val_ref):
    rb = pl.program_id(0)
    r0 = pl.multiple_of(rb * R, R)
    hr = h_ref[pl.ds(r0, R), :]
    hrb = hr.astype(jnp.bfloat16)
    x2rt = _dot_hi(jnp.ones((1, HP), jnp.float32), hr * hr,
                   (((1,), (1,)), ((), ())))
    br = bcol_ref[:, pl.ds(r0, R)]
    rid = r0 + lax.broadcasted_iota(jnp.int32, (1, R), 1)
    w0 = w0_ref[rb, 0]
    nt = nt_ref[rb, 0]

    def body(t, carry):
        bv, bi = carry
        c0 = pl.multiple_of(w0 + t * C, 8)
        hc = h_ref[pl.ds(c0, C), :]
        x2c = jnp.sum(hc * hc, axis=1, keepdims=True)
        hh = lax.dot_general(hc.astype(jnp.bfloat16), hrb,
                             (((1,), (1,)), ((), ())),
                             preferred_element_type=jnp.float32)
        d = (x2c + x2rt) - 2.0 * hh
        bc = brow_ref[pl.ds(c0, C), :]
        cid = c0 + lax.broadcasted_iota(jnp.int32, (C, R), 0)
        ok = (bc == br) & (cid != rid)
        d = jnp.where(ok, d, BIG)
        tvs, tis = [], []
        for _ in range(K):
            m = jnp.min(d, axis=0, keepdims=True)
            im = jnp.min(jnp.where(d == m, cid, jnp.int32(2**30)),
                         axis=0, keepdims=True)
            tvs.append(m)
            tis.append(im)
            d = jnp.where(cid == im, BIG, d)
        cv = jnp.concatenate([bv] + tvs, axis=0)
        ci = jnp.concatenate([bi] + tis, axis=0)
        pos = lax.broadcasted_iota(jnp.int32, (2 * K, R), 0)
        nvs, nis = [], []
        for _ in range(K):
            m = jnp.min(cv, axis=0, keepdims=True)
            pm = jnp.min(jnp.where(cv == m, pos, jnp.int32(99)),
                         axis=0, keepdims=True)
            iv = jnp.sum(jnp.where(pos == pm, ci, 0), axis=0, keepdims=True)
            nvs.append(m)
            nis.append(iv)
            cv = jnp.where(pos == pm, BIG, cv)
        return jnp.concatenate(nvs, axis=0), jnp.concatenate(nis, axis=0)

    bv0 = jnp.full((K, R), BIG, jnp.float32)
    bi0 = jnp.zeros((K, R), jnp.int32)
    bv, bi = lax.fori_loop(0, nt, body, (bv0, bi0))
    idx_ref[...] = bi
    validf = (bv < 1e20).astype(jnp.float32)
    eye = (lax.broadcasted_iota(jnp.int32, (K, K), 0)
           == lax.broadcasted_iota(jnp.int32, (K, K), 1)).astype(jnp.float32)
    val_ref[...] = _dot_hi(validf, eye, (((0,), (0,)), ((), ())))


def _knn(w0, nt, h, brow, bcol):
    smem = pl.BlockSpec(memory_space=pltpu.MemorySpace.SMEM)
    return pl.pallas_call(
        _knn_kernel,
        grid=(NRB,),
        in_specs=[smem, smem, _full((NP, HP)), _full((NP, 1)), _full((1, NP))],
        out_specs=[pl.BlockSpec((K, R), lambda i: (0, i)),
                   pl.BlockSpec((R, K), lambda i: (i, 0))],
        out_shape=[jax.ShapeDtypeStruct((K, NP), jnp.int32),
                   jax.ShapeDtypeStruct((NP, K), jnp.float32)],
    )(w0, nt, h, brow, bcol)


WPK = NW // K
ROWS_W = NP // WPK
NCH = ROWS_W // GCH


def _gather_body(h_hbm, idx_hbm, out_hbm, idx_v, rows_v, sem):
    wid = lax.axis_index("s") * 2 + lax.axis_index("c")
    kk = wid // WPK
    base = (wid % WPK) * ROWS_W
    pltpu.sync_copy(idx_hbm.at[kk, pl.ds(base, ROWS_W)], idx_v)
    cps = [pltpu.async_copy(h_hbm.at[idx_v.at[pl.ds(j * GCH, GCH)]],
                            rows_v.at[pl.ds(j * GCH, GCH)], sem)
           for j in range(NCH)]
    for cp in cps:
        cp.wait()
    pltpu.sync_copy(rows_v, out_hbm.at[pl.ds(kk * NP + base, ROWS_W)])


@functools.cache
def _sc_gather_fn():
    return pl.kernel(
        _gather_body,
        out_type=jax.ShapeDtypeStruct((K * NP, HP), jnp.float32),
        mesh=plsc.VectorSubcoreMesh(core_axis_name="c", subcore_axis_name="s"),
        compiler_params=pltpu.CompilerParams(use_tc_tiling_on_sc=False),
        scratch_types=[
            pltpu.VMEM((ROWS_W,), jnp.int32),
            pltpu.VMEM((ROWS_W, HP), jnp.float32),
            pltpu.SemaphoreType.DMA,
        ],
    )


def _sc_gather(h, idx):
    return _sc_gather_fn()(h, idx)


def _edge_kernel(h_ref, xj_ref, val_ref, w1, b1, w2, b2, o_ref):
    rb = pl.program_id(0)
    r0 = rb * RE
    xi = h_ref[pl.ds(r0, RE), :]
    acc = jnp.zeros((RE, HP), jnp.float32)
    for k in range(K):
        xjk = xj_ref[pl.ds(k * NP + r0, RE), :]
        cat = jnp.concatenate([xi, xjk - xi], axis=1)
        m1 = _elu(_dot(cat, w1[...]) + b1[...])
        m2 = _elu(_dot(m1, w2[...]) + b2[...])
        vk = val_ref[:, k:k + 1]
        acc = acc + m2 * vk
    o_ref[...] = acc


def _edge(h, xj, val, ws):
    return pl.pallas_call(
        _edge_kernel,
        grid=(NP // RE,),
        in_specs=[_full((NP, HP)), _full((K * NP, HP)),
                  pl.BlockSpec((RE, K), lambda i: (i, 0))]
        + [_full(w.shape) for w in ws],
        out_specs=pl.BlockSpec((RE, HP), lambda i: (i, 0)),
        out_shape=jax.ShapeDtypeStruct((NP, HP), jnp.float32),
    )(h, xj, val, *ws)


def _pool_kernel(h_ref, bcol_ref, v1, c1, v2, c2, v3, c3, o_ref, acc_ref):
    rb = pl.program_id(0)
    gids = lax.broadcasted_iota(jnp.int32, (NG, RP), 0)
    s = (gids == bcol_ref[...]).astype(jnp.float32)
    part = _dot_hi(s, h_ref[...], (((1,), (0,)), ((), ())))

    @pl.when(rb == 0)
    def _():
        acc_ref[...] = part

    @pl.when(rb > 0)
    def _():
        acc_ref[...] = acc_ref[...] + part

    @pl.when(rb == NP // RP - 1)
    def _():
        g = acc_ref[...]
        o1 = _elu(_dot(g, v1[...]) + c1[...])
        o2 = _elu(_dot(o1, v2[...]) + c2[...])
        lg = _dot(o2, v3[...]) + c3[...]
        mx = jnp.max(lg, axis=1, keepdims=True)
        ls = jnp.log(jnp.sum(jnp.exp(lg - mx), axis=1, keepdims=True))
        o_ref[...] = lg - mx - ls


def _pool_out(h, bcol, ws):
    return pl.pallas_call(
        _pool_kernel,
        grid=(NP // RP,),
        in_specs=[pl.BlockSpec((RP, HP), lambda i: (i, 0)),
                  pl.BlockSpec((1, RP), lambda i: (0, i))]
        + [_full(w.shape) for w in ws],
        out_specs=pl.BlockSpec((NG, OUT), lambda i: (0, 0)),
        out_shape=jax.ShapeDtypeStruct((NG, OUT), jnp.float32),
        scratch_shapes=[pltpu.VMEM((NG, HP), jnp.float32)],
    )(h, bcol, *ws)


def _pad_w(w, rows, cols):
    return jnp.pad(w, ((0, rows - w.shape[0]), (0, cols - w.shape[1])))


def _pad_b(b, cols):
    return jnp.pad(b, (0, cols - b.shape[0])).reshape(1, cols)


def kernel(x, params, batch):
    batch = batch.astype(jnp.int32)
    xp = jnp.pad(x, ((0, NP - N), (0, 0)))
    bpad = jnp.concatenate([batch, jnp.full((NP - N,), SENT, jnp.int32)])
    brow = bpad.reshape(NP, 1)
    bcol = bpad.reshape(1, NP)
    brs = bpad.reshape(NRB, R)

    w0, nt = _windows(brs, bcol)

    p_in = params["in_net"]
    in_ws = []
    fan = [IN, HP, HP]
    for (w, b), f in zip(p_in, fan):
        in_ws += [_pad_w(w, f, HP), _pad_b(b, HP)]

    edge_ws = []
    for (w1, b1), (w2, b2) in params["edge_nets"]:
        w1p = jnp.zeros((2 * HP, HP), jnp.float32)
        w1p = w1p.at[:HID, :HID].set(w1[:HID])
        w1p = w1p.at[HP:HP + HID, :HID].set(w1[HID:])
        edge_ws.append([w1p, _pad_b(b1, HP),
                        _pad_w(w2, HP, HP), _pad_b(b2, HP)])

    p_out = params["out_net"]
    out_ws = []
    for (w, b), c in zip(p_out, [HP, HP, OUT]):
        out_ws += [_pad_w(w, HP, c), _pad_b(b, c)]

    h = _in_net(xp, in_ws)
    for lws in edge_ws:
        idx, val = _knn(w0, nt, h, brow, bcol)
        xj = _sc_gather(h, idx)
        h = _edge(h, xj, val, lws)
    return _pool_out(h, bcol, out_ws)


# speedup vs baseline: 19.5203x; 1.0044x over previous
import functools

import jax
import jax.numpy as jnp
from jax import lax
from jax.experimental import pallas as pl
from jax.experimental.pallas import tpu as pltpu
from jax.experimental.pallas import tpu_sc as plsc

N = 10000
IN = 128
HID = 20
OUT = 10
K = 4
NG = 1024

HP = 32
R = 128
C = 128
NP = 10240
NRB = NP // R
RX = 1024
RE = 512
RP = 512
SENT = 1 << 20
BIG = 1e30

NW = 32
GCH = 128


def _elu(x):
    return jnp.where(x > 0, x, jnp.exp(jnp.minimum(x, 0.0)) - 1.0)


def _dot(a, b):
    return jnp.dot(a.astype(jnp.bfloat16), b.astype(jnp.bfloat16),
                   preferred_element_type=jnp.float32)


def _dot_hi(a, b, dims):
    return lax.dot_general(a, b, dims, precision=jax.lax.Precision.HIGHEST,
                           preferred_element_type=jnp.float32)


def _in_net_kernel(x_ref, w1, b1, w2, b2, w3, b3, o_ref):
    h = _elu(_dot(x_ref[...], w1[...]) + b1[...])
    h = _elu(_dot(h, w2[...]) + b2[...])
    h = _elu(_dot(h, w3[...]) + b3[...])
    o_ref[...] = h


def _full(shape):
    return pl.BlockSpec(shape, lambda i: (0,) * len(shape))


def _in_net(xp, ws):
    return pl.pallas_call(
        _in_net_kernel,
        grid=(NP // RX,),
        in_specs=[pl.BlockSpec((RX, IN), lambda i: (i, 0))]
        + [_full(w.shape) for w in ws],
        out_specs=pl.BlockSpec((RX, HP), lambda i: (i, 0)),
        out_shape=jax.ShapeDtypeStruct((NP, HP), jnp.float32),
    )(xp, *ws)


def _win_kernel(brs_ref, bcol_ref, w0_ref, nt_ref):
    bf = brs_ref[:, 0:1]
    bl = brs_ref[:, R - 1:R]
    bc = bcol_ref[...]
    starts = jnp.sum((bc < bf).astype(jnp.int32), axis=1, keepdims=True)
    ends = jnp.sum((bc <= bl).astype(jnp.int32), axis=1, keepdims=True)
    w0 = (starts // 8) * 8
    w0_ref[...] = w0
    nt_ref[...] = jnp.minimum((ends - w0 + C - 1) // C, (NP - w0) // C)


def _windows(brs, bcol):
    return pl.pallas_call(
        _win_kernel,
        grid=(1,),
        in_specs=[_full((NRB, R)), _full((1, NP))],
        out_specs=[pl.BlockSpec((NRB, 1), lambda i: (0, 0)),
                   pl.BlockSpec((NRB, 1), lambda i: (0, 0))],
        out_shape=[jax.ShapeDtypeStruct((NRB, 1), jnp.int32),
                   jax.ShapeDtypeStruct((NRB, 1), jnp.int32)],
    )(brs, bcol)


def _knn_kernel(w0_ref, nt_ref, h_ref, brow_ref, bcol_ref, idx_ref, val_ref):
    rb = pl.program_id(0)
    r0 = pl.multiple_of(rb * R, R)
    hr = h_ref[pl.ds(r0, R), :]
    hrb = hr.astype(jnp.bfloat16)
    x2rt = _dot_hi(jnp.ones((1, HP), jnp.float32), hr * hr,
                   (((1,), (1,)), ((), ())))
    br = bcol_ref[:, pl.ds(r0, R)]
    rid = r0 + lax.broadcasted_iota(jnp.int32, (1, R), 1)
    w0 = w0_ref[rb, 0]
    nt = nt_ref[rb, 0]

    def body(t, carry):
        bv, bi = carry
        c0 = pl.multiple_of(w0 + t * C, 8)
        hc = h_ref[pl.ds(c0, C), :]
        x2c = jnp.sum(hc * hc, axis=1, keepdims=True)
        hh = lax.dot_general(hc.astype(jnp.bfloat16), hrb,
                             (((1,), (1,)), ((), ())),
                             preferred_element_type=jnp.float32)
        d = (x2c + x2rt) - 2.0 * hh
        bc = brow_ref[pl.ds(c0, C), :]
        cid = c0 + lax.broadcasted_iota(jnp.int32, (C, R), 0)
        ok = (bc == br) & (cid != rid)
        d = jnp.where(ok, d, BIG)
        tvs, tis = [], []
        for _ in range(K):
            m = jnp.min(d, axis=0, keepdims=True)
            im = jnp.min(jnp.where(d == m, cid, jnp.int32(2**30)),
                         axis=0, keepdims=True)
            tvs.append(m)
            tis.append(im)
            d = jnp.where(cid == im, BIG, d)
        cv = jnp.concatenate([bv] + tvs, axis=0)
        ci = jnp.concatenate([bi] + tis, axis=0)
        pos = lax.broadcasted_iota(jnp.int32, (2 * K, R), 0)
        nvs, nis = [], []
        for _ in range(K):
            m = jnp.min(cv, axis=0, keepdims=True)
            pm = jnp.min(jnp.where(cv == m, pos, jnp.int32(99)),
                         axis=0, keepdims=True)
            iv = jnp.sum(jnp.where(pos == pm, ci, 0), axis=0, keepdims=True)
            nvs.append(m)
            nis.append(iv)
            cv = jnp.where(pos == pm, BIG, cv)
        return jnp.concatenate(nvs, axis=0), jnp.concatenate(nis, axis=0)

    bv0 = jnp.full((K, R), BIG, jnp.float32)
    bi0 = jnp.zeros((K, R), jnp.int32)
    bv, bi = lax.fori_loop(0, nt, body, (bv0, bi0))
    idx_ref[...] = jnp.concatenate(
        [bi, jnp.zeros((8 - K, R), jnp.int32)], axis=0)[None]
    validf = (bv < 1e20).astype(jnp.float32)
    eye = (lax.broadcasted_iota(jnp.int32, (K, K), 0)
           == lax.broadcasted_iota(jnp.int32, (K, K), 1)).astype(jnp.float32)
    val_ref[...] = _dot_hi(validf, eye, (((0,), (0,)), ((), ())))


def _knn(w0, nt, h, brow, bcol):
    smem = pl.BlockSpec(memory_space=pltpu.MemorySpace.SMEM)
    return pl.pallas_call(
        _knn_kernel,
        grid=(NRB,),
        in_specs=[smem, smem, _full((NP, HP)), _full((NP, 1)), _full((1, NP))],
        out_specs=[pl.BlockSpec((1, 8, R), lambda i: (i, 0, 0)),
                   pl.BlockSpec((R, K), lambda i: (i, 0))],
        out_shape=[jax.ShapeDtypeStruct((NRB, 8, R), jnp.int32),
                   jax.ShapeDtypeStruct((NP, K), jnp.float32)],
    )(w0, nt, h, brow, bcol)


WPK = NW // K
ROWS_W = NP // WPK
NCH = ROWS_W // GCH


def _gather_body(h_hbm, idx_hbm, out_hbm, idx_v, rows_v, sem):
    wid = lax.axis_index("s") * 2 + lax.axis_index("c")
    kk = wid // WPK
    rb0 = (wid % WPK) * NCH
    pltpu.sync_copy(idx_hbm.at[pl.ds(rb0, NCH), kk], idx_v)
    cps = [pltpu.async_copy(h_hbm.at[idx_v.at[j]],
                            rows_v.at[pl.ds(j * GCH, GCH)], sem)
           for j in range(NCH)]
    for cp in cps:
        cp.wait()
    pltpu.sync_copy(rows_v, out_hbm.at[pl.ds(kk * NP + rb0 * GCH, ROWS_W)])


@functools.cache
def _sc_gather_fn():
    return pl.kernel(
        _gather_body,
        out_type=jax.ShapeDtypeStruct((K * NP, HP), jnp.float32),
        mesh=plsc.VectorSubcoreMesh(core_axis_name="c", subcore_axis_name="s"),
        compiler_params=pltpu.CompilerParams(use_tc_tiling_on_sc=False),
        scratch_types=[
            pltpu.VMEM((NCH, GCH), jnp.int32),
            pltpu.VMEM((ROWS_W, HP), jnp.float32),
            pltpu.SemaphoreType.DMA,
        ],
    )


def _sc_gather(h, idx):
    return _sc_gather_fn()(h, idx)


def _edge_kernel(h_ref, xj_ref, val_ref, w1, b1, w2, b2, o_ref):
    rb = pl.program_id(0)
    r0 = rb * RE
    xi = h_ref[pl.ds(r0, RE), :]
    acc = jnp.zeros((RE, HP), jnp.float32)
    for k in range(K):
        xjk = xj_ref[pl.ds(k * NP + r0, RE), :]
        cat = jnp.concatenate([xi, xjk - xi], axis=1)
        m1 = _elu(_dot(cat, w1[...]) + b1[...])
        m2 = _elu(_dot(m1, w2[...]) + b2[...])
        vk = val_ref[:, k:k + 1]
        acc = acc + m2 * vk
    o_ref[...] = acc


def _edge(h, xj, val, ws):
    return pl.pallas_call(
        _edge_kernel,
        grid=(NP // RE,),
        in_specs=[_full((NP, HP)), _full((K * NP, HP)),
                  pl.BlockSpec((RE, K), lambda i: (i, 0))]
        + [_full(w.shape) for w in ws],
        out_specs=pl.BlockSpec((RE, HP), lambda i: (i, 0)),
        out_shape=jax.ShapeDtypeStruct((NP, HP), jnp.float32),
    )(h, xj, val, *ws)


def _pool_kernel(h_ref, bcol_ref, v1, c1, v2, c2, v3, c3, o_ref, acc_ref):
    rb = pl.program_id(0)
    gids = lax.broadcasted_iota(jnp.int32, (NG, RP), 0)
    s = (gids == bcol_ref[...]).astype(jnp.float32)
    part = _dot_hi(s, h_ref[...], (((1,), (0,)), ((), ())))

    @pl.when(rb == 0)
    def _():
        acc_ref[...] = part

    @pl.when(rb > 0)
    def _():
        acc_ref[...] = acc_ref[...] + part

    @pl.when(rb == NP // RP - 1)
    def _():
        g = acc_ref[...]
        o1 = _elu(_dot(g, v1[...]) + c1[...])
        o2 = _elu(_dot(o1, v2[...]) + c2[...])
        lg = _dot(o2, v3[...]) + c3[...]
        mx = jnp.max(lg, axis=1, keepdims=True)
        ls = jnp.log(jnp.sum(jnp.exp(lg - mx), axis=1, keepdims=True))
        o_ref[...] = lg - mx - ls


def _pool_out(h, bcol, ws):
    return pl.pallas_call(
        _pool_kernel,
        grid=(NP // RP,),
        in_specs=[pl.BlockSpec((RP, HP), lambda i: (i, 0)),
                  pl.BlockSpec((1, RP), lambda i: (0, i))]
        + [_full(w.shape) for w in ws],
        out_specs=pl.BlockSpec((NG, OUT), lambda i: (0, 0)),
        out_shape=jax.ShapeDtypeStruct((NG, OUT), jnp.float32),
        scratch_shapes=[pltpu.VMEM((NG, HP), jnp.float32)],
    )(h, bcol, *ws)


def _pad_w(w, rows, cols):
    return jnp.pad(w, ((0, rows - w.shape[0]), (0, cols - w.shape[1])))


def _pad_b(b, cols):
    return jnp.pad(b, (0, cols - b.shape[0])).reshape(1, cols)


def kernel(x, params, batch):
    batch = batch.astype(jnp.int32)
    xp = jnp.pad(x, ((0, NP - N), (0, 0)))
    bpad = jnp.concatenate([batch, jnp.full((NP - N,), SENT, jnp.int32)])
    brow = bpad.reshape(NP, 1)
    bcol = bpad.reshape(1, NP)
    brs = bpad.reshape(NRB, R)

    w0, nt = _windows(brs, bcol)

    p_in = params["in_net"]
    in_ws = []
    fan = [IN, HP, HP]
    for (w, b), f in zip(p_in, fan):
        in_ws += [_pad_w(w, f, HP), _pad_b(b, HP)]

    edge_ws = []
    for (w1, b1), (w2, b2) in params["edge_nets"]:
        w1p = jnp.zeros((2 * HP, HP), jnp.float32)
        w1p = w1p.at[:HID, :HID].set(w1[:HID])
        w1p = w1p.at[HP:HP + HID, :HID].set(w1[HID:])
        edge_ws.append([w1p, _pad_b(b1, HP),
                        _pad_w(w2, HP, HP), _pad_b(b2, HP)])

    p_out = params["out_net"]
    out_ws = []
    for (w, b), c in zip(p_out, [HP, HP, OUT]):
        out_ws += [_pad_w(w, HP, c), _pad_b(b, c)]

    h = _in_net(xp, in_ws)
    for lws in edge_ws:
        idx, val = _knn(w0, nt, h, brow, bcol)
        xj = _sc_gather(h, idx)
        h = _edge(h, xj, val, lws)
    return _pool_out(h, bcol, out_ws)


# speedup vs baseline: 21.8317x; 1.1184x over previous
import functools

import jax
import jax.numpy as jnp
from jax import lax
from jax.experimental import pallas as pl
from jax.experimental.pallas import tpu as pltpu
from jax.experimental.pallas import tpu_sc as plsc

N = 10000
IN = 128
HID = 20
OUT = 10
K = 4
NG = 1024

HP = 32
R = 128
C = 192
SUB = 32
NP = 10240
NRB = NP // R
RX = 1024
RE = 512
RP = 512
SENT = 1 << 20
BIG = 1e30

NW = 32
GCH = 128


def _elu(x):
    return jnp.where(x > 0, x, jnp.exp(jnp.minimum(x, 0.0)) - 1.0)


def _dot(a, b):
    return jnp.dot(a.astype(jnp.bfloat16), b.astype(jnp.bfloat16),
                   preferred_element_type=jnp.float32)


def _dot_hi(a, b, dims):
    return lax.dot_general(a, b, dims, precision=jax.lax.Precision.HIGHEST,
                           preferred_element_type=jnp.float32)


def _in_net_kernel(x_ref, w1, b1, w2, b2, w3, b3, o_ref):
    h = _elu(_dot(x_ref[...], w1[...]) + b1[...])
    h = _elu(_dot(h, w2[...]) + b2[...])
    h = _elu(_dot(h, w3[...]) + b3[...])
    o_ref[...] = h


def _full(shape):
    return pl.BlockSpec(shape, lambda i: (0,) * len(shape))


def _in_net(xp, ws):
    return pl.pallas_call(
        _in_net_kernel,
        grid=(NP // RX,),
        in_specs=[pl.BlockSpec((RX, IN), lambda i: (i, 0))]
        + [_full(w.shape) for w in ws],
        out_specs=pl.BlockSpec((RX, HP), lambda i: (i, 0)),
        out_shape=jax.ShapeDtypeStruct((NP, HP), jnp.float32),
    )(xp, *ws)


def _win_kernel(brs_ref, bcol_ref, w0_ref, nt_ref):
    bf = brs_ref[:, 0:1]
    bl = brs_ref[:, R - 1:R]
    bc = bcol_ref[...]
    starts = jnp.sum((bc < bf).astype(jnp.int32), axis=1, keepdims=True)
    ends = jnp.sum((bc <= bl).astype(jnp.int32), axis=1, keepdims=True)
    w0 = (starts // 8) * 8
    w0_ref[...] = w0
    nt_ref[...] = jnp.minimum((ends - w0 + C - 1) // C, (NP - w0) // C)


def _windows(brs, bcol):
    return pl.pallas_call(
        _win_kernel,
        grid=(1,),
        in_specs=[_full((NRB, R)), _full((1, NP))],
        out_specs=[pl.BlockSpec((NRB, 1), lambda i: (0, 0)),
                   pl.BlockSpec((NRB, 1), lambda i: (0, 0))],
        out_shape=[jax.ShapeDtypeStruct((NRB, 1), jnp.int32),
                   jax.ShapeDtypeStruct((NRB, 1), jnp.int32)],
    )(brs, bcol)


def _knn_kernel(w0_ref, nt_ref, h_ref, brow_ref, bcol_ref, idx_ref, val_ref):
    rb = pl.program_id(0)
    r0 = pl.multiple_of(rb * R, R)
    hr = h_ref[pl.ds(r0, R), :]
    hrb = hr.astype(jnp.bfloat16)
    x2rt = _dot_hi(jnp.ones((1, HP), jnp.float32), hr * hr,
                   (((1,), (1,)), ((), ())))
    br = bcol_ref[:, pl.ds(r0, R)]
    rid = r0 + lax.broadcasted_iota(jnp.int32, (1, R), 1)
    w0 = w0_ref[rb, 0]
    nt = nt_ref[rb, 0]

    ones_hp = jnp.ones((1, HP), jnp.float32)

    def body(t, carry):
        bv, bi = carry
        c0 = pl.multiple_of(w0 + t * C, 8)
        hc = h_ref[pl.ds(c0, C), :]
        x2c = _dot_hi(hc * hc, ones_hp, (((1,), (1,)), ((), ())))
        hh = lax.dot_general(hc.astype(jnp.bfloat16), hrb,
                             (((1,), (1,)), ((), ())),
                             preferred_element_type=jnp.float32)
        d = (x2c + x2rt) - 2.0 * hh
        bc = brow_ref[pl.ds(c0, C), :]
        cid = c0 + lax.broadcasted_iota(jnp.int32, (C, R), 0)
        ok = (bc == br) & (cid != rid)
        d = jnp.where(ok, d, BIG)
        cvs, cis = [bv], [bi]
        for s in range(C // SUB):
            ds = d[s * SUB:(s + 1) * SUB]
            cs = cid[s * SUB:(s + 1) * SUB]
            for _ in range(K):
                m = jnp.min(ds, axis=0, keepdims=True)
                im = jnp.min(jnp.where(ds == m, cs, jnp.int32(2**30)),
                             axis=0, keepdims=True)
                cvs.append(m)
                cis.append(im)
                ds = jnp.where(cs == im, BIG, ds)
        cv = jnp.concatenate(cvs, axis=0)
        ci = jnp.concatenate(cis, axis=0)
        ncand = cv.shape[0]
        pos = lax.broadcasted_iota(jnp.int32, (ncand, R), 0)
        nvs, nis = [], []
        for _ in range(K):
            m = jnp.min(cv, axis=0, keepdims=True)
            pm = jnp.min(jnp.where(cv == m, pos, jnp.int32(99)),
                         axis=0, keepdims=True)
            iv = jnp.sum(jnp.where(pos == pm, ci, 0), axis=0, keepdims=True)
            nvs.append(m)
            nis.append(iv)
            cv = jnp.where(pos == pm, BIG, cv)
        return jnp.concatenate(nvs, axis=0), jnp.concatenate(nis, axis=0)

    bv0 = jnp.full((K, R), BIG, jnp.float32)
    bi0 = jnp.zeros((K, R), jnp.int32)
    bv, bi = lax.fori_loop(0, nt, body, (bv0, bi0))
    idx_ref[...] = jnp.concatenate(
        [bi, jnp.zeros((8 - K, R), jnp.int32)], axis=0)[None]
    validf = (bv < 1e20).astype(jnp.float32)
    eye = (lax.broadcasted_iota(jnp.int32, (K, K), 0)
           == lax.broadcasted_iota(jnp.int32, (K, K), 1)).astype(jnp.float32)
    val_ref[...] = _dot_hi(validf, eye, (((0,), (0,)), ((), ())))


def _knn(w0, nt, h, brow, bcol):
    smem = pl.BlockSpec(memory_space=pltpu.MemorySpace.SMEM)
    return pl.pallas_call(
        _knn_kernel,
        grid=(NRB,),
        in_specs=[smem, smem, _full((NP, HP)), _full((NP, 1)), _full((1, NP))],
        out_specs=[pl.BlockSpec((1, 8, R), lambda i: (i, 0, 0)),
                   pl.BlockSpec((R, K), lambda i: (i, 0))],
        out_shape=[jax.ShapeDtypeStruct((NRB, 8, R), jnp.int32),
                   jax.ShapeDtypeStruct((NP, K), jnp.float32)],
    )(w0, nt, h, brow, bcol)


WPK = NW // K
ROWS_W = NP // WPK
NCH = ROWS_W // GCH


def _gather_body(h_hbm, idx_hbm, out_hbm, idx_v, rows_v, sem):
    wid = lax.axis_index("s") * 2 + lax.axis_index("c")
    kk = wid // WPK
    rb0 = (wid % WPK) * NCH
    pltpu.sync_copy(idx_hbm.at[pl.ds(rb0, NCH), kk], idx_v)
    cps = [pltpu.async_copy(h_hbm.at[idx_v.at[j]],
                            rows_v.at[pl.ds(j * GCH, GCH)], sem)
           for j in range(NCH)]
    for cp in cps:
        cp.wait()
    pltpu.sync_copy(rows_v, out_hbm.at[pl.ds(kk * NP + rb0 * GCH, ROWS_W)])


@functools.cache
def _sc_gather_fn():
    return pl.kernel(
        _gather_body,
        out_type=jax.ShapeDtypeStruct((K * NP, HP), jnp.float32),
        mesh=plsc.VectorSubcoreMesh(core_axis_name="c", subcore_axis_name="s"),
        compiler_params=pltpu.CompilerParams(use_tc_tiling_on_sc=False),
        scratch_types=[
            pltpu.VMEM((NCH, GCH), jnp.int32),
            pltpu.VMEM((ROWS_W, HP), jnp.float32),
            pltpu.SemaphoreType.DMA,
        ],
    )


def _sc_gather(h, idx):
    return _sc_gather_fn()(h, idx)


def _edge_kernel(h_ref, xj_ref, val_ref, w1, b1, w2, b2, o_ref):
    rb = pl.program_id(0)
    r0 = rb * RE
    xi = h_ref[pl.ds(r0, RE), :]
    acc = jnp.zeros((RE, HP), jnp.float32)
    for k in range(K):
        xjk = xj_ref[pl.ds(k * NP + r0, RE), :]
        cat = jnp.concatenate([xi, xjk - xi], axis=1)
        m1 = _elu(_dot(cat, w1[...]) + b1[...])
        m2 = _elu(_dot(m1, w2[...]) + b2[...])
        vk = val_ref[:, k:k + 1]
        acc = acc + m2 * vk
    o_ref[...] = acc


def _edge(h, xj, val, ws):
    return pl.pallas_call(
        _edge_kernel,
        grid=(NP // RE,),
        in_specs=[_full((NP, HP)), _full((K * NP, HP)),
                  pl.BlockSpec((RE, K), lambda i: (i, 0))]
        + [_full(w.shape) for w in ws],
        out_specs=pl.BlockSpec((RE, HP), lambda i: (i, 0)),
        out_shape=jax.ShapeDtypeStruct((NP, HP), jnp.float32),
    )(h, xj, val, *ws)


def _pool_kernel(h_ref, bcol_ref, v1, c1, v2, c2, v3, c3, o_ref, acc_ref):
    rb = pl.program_id(0)
    gids = lax.broadcasted_iota(jnp.int32, (NG, RP), 0)
    s = (gids == bcol_ref[...]).astype(jnp.float32)
    part = _dot_hi(s, h_ref[...], (((1,), (0,)), ((), ())))

    @pl.when(rb == 0)
    def _():
        acc_ref[...] = part

    @pl.when(rb > 0)
    def _():
        acc_ref[...] = acc_ref[...] + part

    @pl.when(rb == NP // RP - 1)
    def _():
        g = acc_ref[...]
        o1 = _elu(_dot(g, v1[...]) + c1[...])
        o2 = _elu(_dot(o1, v2[...]) + c2[...])
        lg = _dot(o2, v3[...]) + c3[...]
        mx = jnp.max(lg, axis=1, keepdims=True)
        ls = jnp.log(jnp.sum(jnp.exp(lg - mx), axis=1, keepdims=True))
        o_ref[...] = lg - mx - ls


def _pool_out(h, bcol, ws):
    return pl.pallas_call(
        _pool_kernel,
        grid=(NP // RP,),
        in_specs=[pl.BlockSpec((RP, HP), lambda i: (i, 0)),
                  pl.BlockSpec((1, RP), lambda i: (0, i))]
        + [_full(w.shape) for w in ws],
        out_specs=pl.BlockSpec((NG, OUT), lambda i: (0, 0)),
        out_shape=jax.ShapeDtypeStruct((NG, OUT), jnp.float32),
        scratch_shapes=[pltpu.VMEM((NG, HP), jnp.float32)],
    )(h, bcol, *ws)


def _pad_w(w, rows, cols):
    return jnp.pad(w, ((0, rows - w.shape[0]), (0, cols - w.shape[1])))


def _pad_b(b, cols):
    return jnp.pad(b, (0, cols - b.shape[0])).reshape(1, cols)


def kernel(x, params, batch):
    batch = batch.astype(jnp.int32)
    xp = jnp.pad(x, ((0, NP - N), (0, 0)))
    bpad = jnp.concatenate([batch, jnp.full((NP - N,), SENT, jnp.int32)])
    brow = bpad.reshape(NP, 1)
    bcol = bpad.reshape(1, NP)
    brs = bpad.reshape(NRB, R)

    w0, nt = _windows(brs, bcol)

    p_in = params["in_net"]
    in_ws = []
    fan = [IN, HP, HP]
    for (w, b), f in zip(p_in, fan):
        in_ws += [_pad_w(w, f, HP), _pad_b(b, HP)]

    edge_ws = []
    for (w1, b1), (w2, b2) in params["edge_nets"]:
        w1p = jnp.zeros((2 * HP, HP), jnp.float32)
        w1p = w1p.at[:HID, :HID].set(w1[:HID])
        w1p = w1p.at[HP:HP + HID, :HID].set(w1[HID:])
        edge_ws.append([w1p, _pad_b(b1, HP),
                        _pad_w(w2, HP, HP), _pad_b(b2, HP)])

    p_out = params["out_net"]
    out_ws = []
    for (w, b), c in zip(p_out, [HP, HP, OUT]):
        out_ws += [_pad_w(w, HP, c), _pad_b(b, c)]

    h = _in_net(xp, in_ws)
    for lws in edge_ws:
        idx, val = _knn(w0, nt, h, brow, bcol)
        xj = _sc_gather(h, idx)
        h = _edge(h, xj, val, lws)
    return _pool_out(h, bcol, out_ws)


# speedup vs baseline: 24.9102x; 1.1410x over previous
import functools

import jax
import jax.numpy as jnp
from jax import lax
from jax.experimental import pallas as pl
from jax.experimental.pallas import tpu as pltpu
from jax.experimental.pallas import tpu_sc as plsc

N = 10000
IN = 128
HID = 20
OUT = 10
K = 4
NG = 1024

HP = 32
R = 128
C = 192
SUB = 32
NP = 10240
NRB = NP // R
RX = 1024
RE = 512
RP = 512
SENT = 1 << 20
BIG = 1e30

NW = 32
GCH = 128


def _elu(x):
    return jnp.where(x > 0, x, jnp.exp(jnp.minimum(x, 0.0)) - 1.0)


def _dot(a, b):
    return jnp.dot(a.astype(jnp.bfloat16), b.astype(jnp.bfloat16),
                   preferred_element_type=jnp.float32)


def _dot_hi(a, b, dims):
    return lax.dot_general(a, b, dims,
                           preferred_element_type=jnp.float32)


def _in_net_kernel(x_ref, w1, b1, w2, b2, w3, b3, o_ref):
    h = _elu(_dot(x_ref[...], w1[...]) + b1[...])
    h = _elu(_dot(h, w2[...]) + b2[...])
    h = _elu(_dot(h, w3[...]) + b3[...])
    o_ref[...] = h


def _full(shape):
    return pl.BlockSpec(shape, lambda i: (0,) * len(shape))


def _in_net(xp, ws):
    return pl.pallas_call(
        _in_net_kernel,
        grid=(NP // RX,),
        in_specs=[pl.BlockSpec((RX, IN), lambda i: (i, 0))]
        + [_full(w.shape) for w in ws],
        out_specs=pl.BlockSpec((RX, HP), lambda i: (i, 0)),
        out_shape=jax.ShapeDtypeStruct((NP, HP), jnp.float32),
    )(xp, *ws)


def _win_kernel(brs_ref, bcol_ref, w0_ref, nt_ref):
    bf = brs_ref[:, 0:1]
    bl = brs_ref[:, R - 1:R]
    bc = bcol_ref[...]
    starts = jnp.sum((bc < bf).astype(jnp.int32), axis=1, keepdims=True)
    ends = jnp.sum((bc <= bl).astype(jnp.int32), axis=1, keepdims=True)
    w0 = (starts // 8) * 8
    w0_ref[...] = w0
    nt_ref[...] = jnp.minimum((ends - w0 + C - 1) // C, (NP - w0) // C)


def _windows(brs, bcol):
    return pl.pallas_call(
        _win_kernel,
        grid=(1,),
        in_specs=[_full((NRB, R)), _full((1, NP))],
        out_specs=[pl.BlockSpec((NRB, 1), lambda i: (0, 0)),
                   pl.BlockSpec((NRB, 1), lambda i: (0, 0))],
        out_shape=[jax.ShapeDtypeStruct((NRB, 1), jnp.int32),
                   jax.ShapeDtypeStruct((NRB, 1), jnp.int32)],
    )(brs, bcol)


def _knn_kernel(w0_ref, nt_ref, h_ref, brow_ref, bcol_ref, idx_ref, val_ref):
    rb = pl.program_id(0)
    r0 = pl.multiple_of(rb * R, R)
    hr = h_ref[pl.ds(r0, R), :]
    hrb = hr.astype(jnp.bfloat16)
    x2rt = _dot_hi(jnp.ones((1, HP), jnp.float32), hr * hr,
                   (((1,), (1,)), ((), ())))
    br = bcol_ref[:, pl.ds(r0, R)]
    rid = r0 + lax.broadcasted_iota(jnp.int32, (1, R), 1)
    w0 = w0_ref[rb, 0]
    nt = nt_ref[rb, 0]

    ones_hp = jnp.ones((1, HP), jnp.float32)

    def body(t, carry):
        bv, bi = carry
        c0 = pl.multiple_of(w0 + t * C, 8)
        hc = h_ref[pl.ds(c0, C), :]
        x2c = _dot_hi(hc * hc, ones_hp, (((1,), (1,)), ((), ())))
        hh = lax.dot_general(hc.astype(jnp.bfloat16), hrb,
                             (((1,), (1,)), ((), ())),
                             preferred_element_type=jnp.float32)
        d = (x2c + x2rt) - 2.0 * hh
        bc = brow_ref[pl.ds(c0, C), :]
        cid = c0 + lax.broadcasted_iota(jnp.int32, (C, R), 0)
        ok = (bc == br) & (cid != rid)
        d = jnp.where(ok, d, BIG)
        cvs, cis = [bv], [bi]
        for s in range(C // SUB):
            ds = d[s * SUB:(s + 1) * SUB]
            cs = cid[s * SUB:(s + 1) * SUB]
            for _ in range(K):
                m = jnp.min(ds, axis=0, keepdims=True)
                im = jnp.min(jnp.where(ds == m, cs, jnp.int32(2**30)),
                             axis=0, keepdims=True)
                cvs.append(m)
                cis.append(im)
                ds = jnp.where(cs == im, BIG, ds)
        cv = jnp.concatenate(cvs, axis=0)
        ci = jnp.concatenate(cis, axis=0)
        ncand = cv.shape[0]
        pos = lax.broadcasted_iota(jnp.int32, (ncand, R), 0)
        nvs, nis = [], []
        for _ in range(K):
            m = jnp.min(cv, axis=0, keepdims=True)
            pm = jnp.min(jnp.where(cv == m, pos, jnp.int32(99)),
                         axis=0, keepdims=True)
            iv = jnp.sum(jnp.where(pos == pm, ci, 0), axis=0, keepdims=True)
            nvs.append(m)
            nis.append(iv)
            cv = jnp.where(pos == pm, BIG, cv)
        return jnp.concatenate(nvs, axis=0), jnp.concatenate(nis, axis=0)

    bv0 = jnp.full((K, R), BIG, jnp.float32)
    bi0 = jnp.zeros((K, R), jnp.int32)
    bv, bi = lax.fori_loop(0, nt, body, (bv0, bi0))
    idx_ref[...] = jnp.concatenate(
        [bi, jnp.zeros((8 - K, R), jnp.int32)], axis=0)[None]
    validf = (bv < 1e20).astype(jnp.float32)
    eye = (lax.broadcasted_iota(jnp.int32, (K, K), 0)
           == lax.broadcasted_iota(jnp.int32, (K, K), 1)).astype(jnp.float32)
    val_ref[...] = _dot_hi(validf, eye, (((0,), (0,)), ((), ())))


def _knn(w0, nt, h, brow, bcol):
    smem = pl.BlockSpec(memory_space=pltpu.MemorySpace.SMEM)
    return pl.pallas_call(
        _knn_kernel,
        grid=(NRB,),
        in_specs=[smem, smem, _full((NP, HP)), _full((NP, 1)), _full((1, NP))],
        out_specs=[pl.BlockSpec((1, 8, R), lambda i: (i, 0, 0)),
                   pl.BlockSpec((R, K), lambda i: (i, 0))],
        out_shape=[jax.ShapeDtypeStruct((NRB, 8, R), jnp.int32),
                   jax.ShapeDtypeStruct((NP, K), jnp.float32)],
    )(w0, nt, h, brow, bcol)


WPK = NW // K
ROWS_W = NP // WPK
NCH = ROWS_W // GCH


def _gather_body(h_hbm, idx_hbm, out_hbm, idx_v, rows_v, sem):
    wid = lax.axis_index("s") * 2 + lax.axis_index("c")
    kk = wid // WPK
    rb0 = (wid % WPK) * NCH
    pltpu.sync_copy(idx_hbm.at[pl.ds(rb0, NCH), kk], idx_v)
    cps = [pltpu.async_copy(h_hbm.at[idx_v.at[j]],
                            rows_v.at[pl.ds(j * GCH, GCH)], sem)
           for j in range(NCH)]
    for cp in cps:
        cp.wait()
    pltpu.sync_copy(rows_v, out_hbm.at[pl.ds(kk * NP + rb0 * GCH, ROWS_W)])


@functools.cache
def _sc_gather_fn():
    return pl.kernel(
        _gather_body,
        out_type=jax.ShapeDtypeStruct((K * NP, HP), jnp.float32),
        mesh=plsc.VectorSubcoreMesh(core_axis_name="c", subcore_axis_name="s"),
        compiler_params=pltpu.CompilerParams(use_tc_tiling_on_sc=False),
        scratch_types=[
            pltpu.VMEM((NCH, GCH), jnp.int32),
            pltpu.VMEM((ROWS_W, HP), jnp.float32),
            pltpu.SemaphoreType.DMA,
        ],
    )


def _sc_gather(h, idx):
    return _sc_gather_fn()(h, idx)


def _edge_kernel(h_ref, xj_ref, val_ref, w1, b1, w2, b2, o_ref):
    rb = pl.program_id(0)
    r0 = rb * RE
    xi = h_ref[pl.ds(r0, RE), :]
    acc = jnp.zeros((RE, HP), jnp.float32)
    for k in range(K):
        xjk = xj_ref[pl.ds(k * NP + r0, RE), :]
        cat = jnp.concatenate([xi, xjk - xi], axis=1)
        m1 = _elu(_dot(cat, w1[...]) + b1[...])
        m2 = _elu(_dot(m1, w2[...]) + b2[...])
        vk = val_ref[:, k:k + 1]
        acc = acc + m2 * vk
    o_ref[...] = acc


def _edge(h, xj, val, ws):
    return pl.pallas_call(
        _edge_kernel,
        grid=(NP // RE,),
        in_specs=[_full((NP, HP)), _full((K * NP, HP)),
                  pl.BlockSpec((RE, K), lambda i: (i, 0))]
        + [_full(w.shape) for w in ws],
        out_specs=pl.BlockSpec((RE, HP), lambda i: (i, 0)),
        out_shape=jax.ShapeDtypeStruct((NP, HP), jnp.float32),
    )(h, xj, val, *ws)


def _pool_kernel(h_ref, bcol_ref, v1, c1, v2, c2, v3, c3, o_ref, acc_ref):
    rb = pl.program_id(0)
    gids = lax.broadcasted_iota(jnp.int32, (NG, RP), 0)
    s = (gids == bcol_ref[...]).astype(jnp.float32)
    part = _dot_hi(s, h_ref[...], (((1,), (0,)), ((), ())))

    @pl.when(rb == 0)
    def _():
        acc_ref[...] = part

    @pl.when(rb > 0)
    def _():
        acc_ref[...] = acc_ref[...] + part

    @pl.when(rb == NP // RP - 1)
    def _():
        g = acc_ref[...]
        o1 = _elu(_dot(g, v1[...]) + c1[...])
        o2 = _elu(_dot(o1, v2[...]) + c2[...])
        lg = _dot(o2, v3[...]) + c3[...]
        mx = jnp.max(lg, axis=1, keepdims=True)
        ls = jnp.log(jnp.sum(jnp.exp(lg - mx), axis=1, keepdims=True))
        o_ref[...] = lg - mx - ls


def _pool_out(h, bcol, ws):
    return pl.pallas_call(
        _pool_kernel,
        grid=(NP // RP,),
        in_specs=[pl.BlockSpec((RP, HP), lambda i: (i, 0)),
                  pl.BlockSpec((1, RP), lambda i: (0, i))]
        + [_full(w.shape) for w in ws],
        out_specs=pl.BlockSpec((NG, OUT), lambda i: (0, 0)),
        out_shape=jax.ShapeDtypeStruct((NG, OUT), jnp.float32),
        scratch_shapes=[pltpu.VMEM((NG, HP), jnp.float32)],
    )(h, bcol, *ws)


def _pad_w(w, rows, cols):
    return jnp.pad(w, ((0, rows - w.shape[0]), (0, cols - w.shape[1])))


def _pad_b(b, cols):
    return jnp.pad(b, (0, cols - b.shape[0])).reshape(1, cols)


def kernel(x, params, batch):
    batch = batch.astype(jnp.int32)
    xp = jnp.pad(x, ((0, NP - N), (0, 0)))
    bpad = jnp.concatenate([batch, jnp.full((NP - N,), SENT, jnp.int32)])
    brow = bpad.reshape(NP, 1)
    bcol = bpad.reshape(1, NP)
    brs = bpad.reshape(NRB, R)

    w0, nt = _windows(brs, bcol)

    p_in = params["in_net"]
    in_ws = []
    fan = [IN, HP, HP]
    for (w, b), f in zip(p_in, fan):
        in_ws += [_pad_w(w, f, HP), _pad_b(b, HP)]

    edge_ws = []
    for (w1, b1), (w2, b2) in params["edge_nets"]:
        w1p = jnp.zeros((2 * HP, HP), jnp.float32)
        w1p = w1p.at[:HID, :HID].set(w1[:HID])
        w1p = w1p.at[HP:HP + HID, :HID].set(w1[HID:])
        edge_ws.append([w1p, _pad_b(b1, HP),
                        _pad_w(w2, HP, HP), _pad_b(b2, HP)])

    p_out = params["out_net"]
    out_ws = []
    for (w, b), c in zip(p_out, [HP, HP, OUT]):
        out_ws += [_pad_w(w, HP, c), _pad_b(b, c)]

    h = _in_net(xp, in_ws)
    for lws in edge_ws:
        idx, val = _knn(w0, nt, h, brow, bcol)
        xj = _sc_gather(h, idx)
        h = _edge(h, xj, val, lws)
    return _pool_out(h, bcol, out_ws)
